# baseline (speedup 1.0000x reference)
import functools

import jax
import jax.numpy as jnp
import numpy as np
from jax.experimental import pallas as pl
from jax.experimental.pallas import tpu as pltpu

B, S, IN_DIM, FIELD, POS, NE = 2, 4096, 768, 256, 64, 512
TOTAL = POS + IN_DIM + 1
BS = 512
KC = 640
f32, bf16, i32 = jnp.float32, jnp.bfloat16, jnp.int32


def _gelu_exact(xv):
    half = xv * np.float32(0.5)
    w = (-xv) * np.float32(0.70710676908493042)
    x2v = w * w
    p = x2v * np.float32(7.85386146e-05) + np.float32(-0.000801019371)
    p = p * x2v + np.float32(0.00518832775)
    p = p * x2v + np.float32(-0.0268538129)
    p = p * x2v + np.float32(0.112835854)
    p = p * x2v + np.float32(-0.37612626)
    p = p * x2v + np.float32(1.12837911)
    res_lt1 = np.float32(1.0) - w * p
    aw = jnp.abs(w)
    q = jnp.exp(-x2v) * (np.float32(1.0) / aw)
    r = np.float32(1.0) / x2v
    p2 = r * np.float32(0.0232682) + np.float32(-0.138703942)
    p2 = p2 * r + np.float32(0.368742466)
    p2 = p2 * r + np.float32(-0.582473278)
    p2 = p2 * r + np.float32(0.621000469)
    p2 = p2 * r + np.float32(-0.494451523)
    p2 = p2 * r + np.float32(0.340488)
    p2 = p2 * r + np.float32(-0.274112701)
    p2 = p2 * r + np.float32(0.563825965)
    p3 = r * np.float32(-10.477664) + np.float32(12.9772)
    p3 = p3 * r + np.float32(-7.49551868)
    p3 = p3 * r + np.float32(2.92101908)
    p3 = p3 * r + np.float32(-1.01526523)
    p3 = p3 * r + np.float32(0.42184633)
    p3 = p3 * r + np.float32(-0.282076746)
    p3 = p3 * r + np.float32(0.564189494)
    polysel = jnp.where(aw < np.float32(2.0), p2, p3)
    val = q * polysel
    val = jnp.where(-x2v < np.float32(-88.7228394), np.float32(0.0), val)
    res_ge1 = jnp.where(w < np.float32(0.0), np.float32(2.0) - val, val)
    erfc_v = jnp.where(aw < np.float32(1.0), res_lt1, res_ge1)
    return half * erfc_v


def _ef_rows(xv, w1, b1, w2, b2, w3, b3):
    fb = (jnp.dot(xv, w1) + b1).astype(bf16)
    hpre = jax.lax.dot_general(fb, w2, (((1,), (0,)), ((), ())),
                               preferred_element_type=f32) + b2
    h = _gelu_exact(hpre)
    return jnp.dot(h, w3) + b3


def _k1_kernel(xr, w1r, b1r, w2r, b2r, w3r, b3r, impr):
    ef = _ef_rows(xr[...], w1r[...], b1r[...], w2r[...], b2r[...], w3r[...], b3r[...])
    sq = ef * ef
    sq = jnp.concatenate([sq, jnp.zeros((sq.shape[0], 7 * 128 - TOTAL), f32)], axis=1)
    acc = sq[:, 0:128]
    for c in range(1, 7):
        acc = acc + sq[:, 128 * c:128 * (c + 1)]
    n = 128
    while n > 1:
        n //= 2
        acc = acc[:, :n] + acc[:, n:2 * n]
    impr[...] = jnp.sqrt(acc)


def _run_k1(x2, W_in, b_in, W_f1, b_f1, W_f2, b_f2):
    return pl.pallas_call(
        _k1_kernel, grid=(B * S // BS,),
        in_specs=[pl.BlockSpec((BS, IN_DIM), lambda i: (i, 0)),
                  pl.BlockSpec(W_in.shape, lambda i: (0, 0)),
                  pl.BlockSpec(b_in.shape, lambda i: (0,)),
                  pl.BlockSpec(W_f1.shape, lambda i: (0, 0)),
                  pl.BlockSpec(b_f1.shape, lambda i: (0,)),
                  pl.BlockSpec(W_f2.shape, lambda i: (0, 0)),
                  pl.BlockSpec(b_f2.shape, lambda i: (0,))],
        out_specs=pl.BlockSpec((BS, 1), lambda i: (i, 0)),
        out_shape=jax.ShapeDtypeStruct((B * S, 1), f32),
    )(x2, W_in, b_in, W_f1, b_f1, W_f2, b_f2)


def _k2_kernel(impr, idxr):
    imp = impr[...]
    m = jnp.max(imp, axis=-1, keepdims=True)
    e = jnp.exp(imp - m)
    sm = e / jnp.sum(e, axis=-1, keepdims=True)
    vb = jax.lax.bitcast_convert_type(sm, i32)
    def body(_, lohi):
        lo, hi = lohi
        mid = lo + jax.lax.div(hi - lo, jnp.int32(2))
        cnt = jnp.sum((vb > mid).astype(i32), axis=-1, keepdims=True)
        small = cnt < NE
        return jnp.where(small, lo, mid + 1), jnp.where(small, mid, hi)
    lo0 = jnp.zeros((B, 1), i32)
    hi0 = jnp.full((B, 1), jnp.int32(0x7F800000))
    lo, hi = jax.lax.fori_loop(0, 31, body, (lo0, hi0))
    tau = hi
    mask = (vb >= tau).astype(i32)
    cum = mask
    sh = 1
    while sh < S:
        cum = cum + jnp.concatenate(
            [jnp.zeros((B, sh), i32), cum[:, :S - sh]], axis=1)
        sh *= 2
    pos = cum - 1
    s_iota = jax.lax.broadcasted_iota(i32, (B, S), 1)
    k_col = jax.lax.broadcasted_iota(i32, (KC, S), 0)
    r_row = jax.lax.broadcasted_iota(i32, (1, NE), 1)
    for b in range(B):
        posb = pos[b:b + 1]
        maskb = mask[b:b + 1]
        vbb = vb[b:b + 1]
        sb = s_iota[b:b + 1]
        sel = jnp.where((k_col == posb) & (maskb > 0), jnp.int32(1), jnp.int32(0))
        cv = jnp.sum(sel * vbb, axis=1, keepdims=True)
        cs = jnp.sum(sel * sb, axis=1, keepdims=True)
        ncand = jnp.sum(maskb)
        rank = jnp.zeros((KC, 1), i32)
        for jc in range(S // BS):
            vj = vbb[:, jc * BS:(jc + 1) * BS]
            sj = sb[:, jc * BS:(jc + 1) * BS]
            gt = (vj > cv).astype(i32)
            tie = ((vj == cv) & (sj < cs)).astype(i32)
            rank = rank + jnp.sum(gt + tie, axis=1, keepdims=True)
        kvalid = jax.lax.broadcasted_iota(i32, (KC, 1), 0) < ncand
        rank = jnp.where(kvalid & (rank < NE), rank, jnp.int32(2 * S))
        eqr = jnp.where(rank == r_row, jnp.int32(1), jnp.int32(0))
        idxr[b, :] = jnp.sum(eqr * cs, axis=0)


def _run_k2(imp):
    return pl.pallas_call(
        _k2_kernel,
        in_specs=[pl.BlockSpec((B, S), lambda: (0, 0))],
        out_specs=pl.BlockSpec((B, NE), lambda: (0, 0)),
        out_shape=jax.ShapeDtypeStruct((B, NE), i32),
    )(imp)


def _k3_kernel(sidx_ref, x_ref, out_ref):
    out_ref[...] = x_ref[...]


def _run_k3(flat_idx, x2):
    x3 = x2.reshape(B * S, 1, IN_DIM)
    grid_spec = pltpu.PrefetchScalarGridSpec(
        num_scalar_prefetch=1,
        grid=(B * NE,),
        in_specs=[pl.BlockSpec((1, 1, IN_DIM), lambda g, sidx: (sidx[g], 0, 0))],
        out_specs=pl.BlockSpec((1, 1, IN_DIM), lambda g, sidx: (g, 0, 0)),
    )
    out = pl.pallas_call(
        _k3_kernel, grid_spec=grid_spec,
        out_shape=jax.ShapeDtypeStruct((B * NE, 1, IN_DIM), f32),
    )(flat_idx, x3)
    return out.reshape(B * NE, IN_DIM)


def _k4_kernel(xr, w1r, b1r, w2r, b2r, w3r, b3r, wspr, bspr, posr, str_, wtr):
    ef = _ef_rows(xr[...], w1r[...], b1r[...], w2r[...], b2r[...], w3r[...], b3r[...])
    posr[...] = ef[:, :POS]
    state = ef[:, POS:POS + FIELD]
    wtr[...] = ef[:, TOTAL - 1:TOTAL]
    str_[...] = jnp.dot(state, wspr[...]) + bspr[...]


def _run_k4(x_sel, W_in, b_in, W_f1, b_f1, W_f2, b_f2, W_sp, b_sp):
    return pl.pallas_call(
        _k4_kernel, grid=(B * NE // BS,),
        in_specs=[pl.BlockSpec((BS, IN_DIM), lambda i: (i, 0)),
                  pl.BlockSpec(W_in.shape, lambda i: (0, 0)),
                  pl.BlockSpec(b_in.shape, lambda i: (0,)),
                  pl.BlockSpec(W_f1.shape, lambda i: (0, 0)),
                  pl.BlockSpec(b_f1.shape, lambda i: (0,)),
                  pl.BlockSpec(W_f2.shape, lambda i: (0, 0)),
                  pl.BlockSpec(b_f2.shape, lambda i: (0,)),
                  pl.BlockSpec(W_sp.shape, lambda i: (0, 0)),
                  pl.BlockSpec(b_sp.shape, lambda i: (0,))],
        out_specs=[pl.BlockSpec((BS, POS), lambda i: (i, 0)),
                   pl.BlockSpec((BS, IN_DIM), lambda i: (i, 0)),
                   pl.BlockSpec((BS, 1), lambda i: (i, 0))],
        out_shape=[jax.ShapeDtypeStruct((B * NE, POS), f32),
                   jax.ShapeDtypeStruct((B * NE, IN_DIM), f32),
                   jax.ShapeDtypeStruct((B * NE, 1), f32)],
    )(x_sel, W_in, b_in, W_f1, b_f1, W_f2, b_f2, W_sp, b_sp)


def kernel(x, W_in, b_in, W_f1, b_f1, W_f2, b_f2, W_sp, b_sp):
    x2 = x.reshape(B * S, IN_DIM)
    imp = _run_k1(x2, W_in, b_in, W_f1, b_f1, W_f2, b_f2).reshape(B, S)
    idx = _run_k2(imp)
    flat_idx = (idx + jnp.arange(B, dtype=i32)[:, None] * S).reshape(-1)
    x_sel = _run_k3(flat_idx, x2)
    positions, states, weights = _run_k4(
        x_sel, W_in, b_in, W_f1, b_f1, W_f2, b_f2, W_sp, b_sp)
    return (positions.reshape(B, NE, POS),
            states.reshape(B, NE, IN_DIM),
            weights.reshape(B, NE, 1))


# speedup vs baseline: 2.5085x; 2.5085x over previous
import functools

import jax
import jax.numpy as jnp
import numpy as np
from jax.experimental import pallas as pl
from jax.experimental.pallas import tpu as pltpu

B, S, IN_DIM, FIELD, POS, NE = 2, 4096, 768, 256, 64, 512
TOTAL = POS + IN_DIM + 1
BS = 512
KC = 640
f32, bf16, i32 = jnp.float32, jnp.bfloat16, jnp.int32


def _gelu_exact(xv):
    half = xv * np.float32(0.5)
    w = (-xv) * np.float32(0.70710676908493042)
    x2v = w * w
    p = x2v * np.float32(7.85386146e-05) + np.float32(-0.000801019371)
    p = p * x2v + np.float32(0.00518832775)
    p = p * x2v + np.float32(-0.0268538129)
    p = p * x2v + np.float32(0.112835854)
    p = p * x2v + np.float32(-0.37612626)
    p = p * x2v + np.float32(1.12837911)
    res_lt1 = np.float32(1.0) - w * p
    aw = jnp.abs(w)
    q = jnp.exp(-x2v) * (np.float32(1.0) / aw)
    r = np.float32(1.0) / x2v
    p2 = r * np.float32(0.0232682) + np.float32(-0.138703942)
    p2 = p2 * r + np.float32(0.368742466)
    p2 = p2 * r + np.float32(-0.582473278)
    p2 = p2 * r + np.float32(0.621000469)
    p2 = p2 * r + np.float32(-0.494451523)
    p2 = p2 * r + np.float32(0.340488)
    p2 = p2 * r + np.float32(-0.274112701)
    p2 = p2 * r + np.float32(0.563825965)
    p3 = r * np.float32(-10.477664) + np.float32(12.9772)
    p3 = p3 * r + np.float32(-7.49551868)
    p3 = p3 * r + np.float32(2.92101908)
    p3 = p3 * r + np.float32(-1.01526523)
    p3 = p3 * r + np.float32(0.42184633)
    p3 = p3 * r + np.float32(-0.282076746)
    p3 = p3 * r + np.float32(0.564189494)
    polysel = jnp.where(aw < np.float32(2.0), p2, p3)
    val = q * polysel
    val = jnp.where(-x2v < np.float32(-88.7228394), np.float32(0.0), val)
    res_ge1 = jnp.where(w < np.float32(0.0), np.float32(2.0) - val, val)
    erfc_v = jnp.where(aw < np.float32(1.0), res_lt1, res_ge1)
    return half * erfc_v


def _ef_rows(xv, w1, b1, w2, b2, w3, b3):
    fb = (jnp.dot(xv, w1) + b1).astype(bf16)
    hpre = jax.lax.dot_general(fb, w2, (((1,), (0,)), ((), ())),
                               preferred_element_type=f32) + b2
    h = _gelu_exact(hpre)
    return jnp.dot(h, w3) + b3


def _k1_kernel(xr, w1r, b1r, w2r, b2r, w3r, b3r, impr):
    ef = _ef_rows(xr[...], w1r[...], b1r[...], w2r[...], b2r[...], w3r[...], b3r[...])
    sq = ef * ef
    sq = jnp.concatenate([sq, jnp.zeros((sq.shape[0], 7 * 128 - TOTAL), f32)], axis=1)
    acc = sq[:, 0:128]
    for c in range(1, 7):
        acc = acc + sq[:, 128 * c:128 * (c + 1)]
    n = 128
    while n > 1:
        n //= 2
        acc = acc[:, :n] + acc[:, n:2 * n]
    impr[...] = jnp.sqrt(acc)


def _run_k1(x2, W_in, b_in, W_f1, b_f1, W_f2, b_f2):
    return pl.pallas_call(
        _k1_kernel, grid=(B * S // BS,),
        in_specs=[pl.BlockSpec((BS, IN_DIM), lambda i: (i, 0)),
                  pl.BlockSpec(W_in.shape, lambda i: (0, 0)),
                  pl.BlockSpec(b_in.shape, lambda i: (0,)),
                  pl.BlockSpec(W_f1.shape, lambda i: (0, 0)),
                  pl.BlockSpec(b_f1.shape, lambda i: (0,)),
                  pl.BlockSpec(W_f2.shape, lambda i: (0, 0)),
                  pl.BlockSpec(b_f2.shape, lambda i: (0,))],
        out_specs=pl.BlockSpec((BS, 1), lambda i: (i, 0)),
        out_shape=jax.ShapeDtypeStruct((B * S, 1), f32),
    )(x2, W_in, b_in, W_f1, b_f1, W_f2, b_f2)


def _k2_kernel(impr, idxr):
    imp = impr[...]
    m = jnp.max(imp, axis=-1, keepdims=True)
    e = jnp.exp(imp - m)
    sm = e / jnp.sum(e, axis=-1, keepdims=True)
    vb = jax.lax.bitcast_convert_type(sm, i32)
    def body(_, lohi):
        lo, hi = lohi
        mid = lo + jax.lax.div(hi - lo, jnp.int32(2))
        cnt = jnp.sum((vb > mid).astype(i32), axis=-1, keepdims=True)
        small = cnt < NE
        return jnp.where(small, lo, mid + 1), jnp.where(small, mid, hi)
    lo0 = jnp.zeros((B, 1), i32)
    hi0 = jnp.full((B, 1), jnp.int32(0x7F800000))
    lo, hi = jax.lax.fori_loop(0, 31, body, (lo0, hi0))
    tau = hi
    mask = (vb >= tau).astype(i32)
    cum = mask
    sh = 1
    while sh < S:
        cum = cum + jnp.concatenate(
            [jnp.zeros((B, sh), i32), cum[:, :S - sh]], axis=1)
        sh *= 2
    pos = cum - 1
    s_iota = jax.lax.broadcasted_iota(i32, (B, S), 1)
    k_col = jax.lax.broadcasted_iota(i32, (KC, S), 0)
    r_row = jax.lax.broadcasted_iota(i32, (1, NE), 1)
    for b in range(B):
        posb = pos[b:b + 1]
        maskb = mask[b:b + 1]
        vbb = vb[b:b + 1]
        sb = s_iota[b:b + 1]
        sel = jnp.where((k_col == posb) & (maskb > 0), jnp.int32(1), jnp.int32(0))
        cv = jnp.sum(sel * vbb, axis=1, keepdims=True)
        cs = jnp.sum(sel * sb, axis=1, keepdims=True)
        ncand = jnp.sum(maskb)
        rank = jnp.zeros((KC, 1), i32)
        for jc in range(S // BS):
            vj = vbb[:, jc * BS:(jc + 1) * BS]
            sj = sb[:, jc * BS:(jc + 1) * BS]
            gt = (vj > cv).astype(i32)
            tie = ((vj == cv) & (sj < cs)).astype(i32)
            rank = rank + jnp.sum(gt + tie, axis=1, keepdims=True)
        kvalid = jax.lax.broadcasted_iota(i32, (KC, 1), 0) < ncand
        rank = jnp.where(kvalid & (rank < NE), rank, jnp.int32(2 * S))
        eqr = jnp.where(rank == r_row, jnp.int32(1), jnp.int32(0))
        idxr[b, :] = jnp.sum(eqr * cs, axis=0)


def _run_k2(imp):
    return pl.pallas_call(
        _k2_kernel,
        in_specs=[pl.BlockSpec((B, S), lambda: (0, 0))],
        out_specs=pl.BlockSpec((B, NE), lambda: (0, 0)),
        out_shape=jax.ShapeDtypeStruct((B, NE), i32),
    )(imp)


GB = 8


def _k3_kernel(sidx_ref, *refs):
    out_ref = refs[-1]
    for i in range(GB):
        out_ref[0, i, :] = refs[i][0, 0, :]


def _run_k3(flat_idx, x2):
    x3 = x2.reshape(B * S, 1, IN_DIM)
    grid_spec = pltpu.PrefetchScalarGridSpec(
        num_scalar_prefetch=1,
        grid=(B * NE // GB,),
        in_specs=[pl.BlockSpec((1, 1, IN_DIM),
                               (lambda i: (lambda g, sidx: (sidx[GB * g + i], 0, 0)))(i))
                  for i in range(GB)],
        out_specs=pl.BlockSpec((1, GB, IN_DIM), lambda g, sidx: (g, 0, 0)),
    )
    out = pl.pallas_call(
        _k3_kernel, grid_spec=grid_spec,
        out_shape=jax.ShapeDtypeStruct((B * NE // GB, GB, IN_DIM), f32),
    )(flat_idx, *([x3] * GB))
    return out.reshape(B * NE, IN_DIM)


def _k4_kernel(xr, w1r, b1r, w2r, b2r, w3r, b3r, wspr, bspr, posr, str_, wtr):
    ef = _ef_rows(xr[...], w1r[...], b1r[...], w2r[...], b2r[...], w3r[...], b3r[...])
    posr[...] = ef[:, :POS]
    state = ef[:, POS:POS + FIELD]
    wtr[...] = ef[:, TOTAL - 1:TOTAL]
    str_[...] = jnp.dot(state, wspr[...]) + bspr[...]


def _run_k4(x_sel, W_in, b_in, W_f1, b_f1, W_f2, b_f2, W_sp, b_sp):
    return pl.pallas_call(
        _k4_kernel, grid=(B * NE // BS,),
        in_specs=[pl.BlockSpec((BS, IN_DIM), lambda i: (i, 0)),
                  pl.BlockSpec(W_in.shape, lambda i: (0, 0)),
                  pl.BlockSpec(b_in.shape, lambda i: (0,)),
                  pl.BlockSpec(W_f1.shape, lambda i: (0, 0)),
                  pl.BlockSpec(b_f1.shape, lambda i: (0,)),
                  pl.BlockSpec(W_f2.shape, lambda i: (0, 0)),
                  pl.BlockSpec(b_f2.shape, lambda i: (0,)),
                  pl.BlockSpec(W_sp.shape, lambda i: (0, 0)),
                  pl.BlockSpec(b_sp.shape, lambda i: (0,))],
        out_specs=[pl.BlockSpec((BS, POS), lambda i: (i, 0)),
                   pl.BlockSpec((BS, IN_DIM), lambda i: (i, 0)),
                   pl.BlockSpec((BS, 1), lambda i: (i, 0))],
        out_shape=[jax.ShapeDtypeStruct((B * NE, POS), f32),
                   jax.ShapeDtypeStruct((B * NE, IN_DIM), f32),
                   jax.ShapeDtypeStruct((B * NE, 1), f32)],
    )(x_sel, W_in, b_in, W_f1, b_f1, W_f2, b_f2, W_sp, b_sp)


def kernel(x, W_in, b_in, W_f1, b_f1, W_f2, b_f2, W_sp, b_sp):
    x2 = x.reshape(B * S, IN_DIM)
    imp = _run_k1(x2, W_in, b_in, W_f1, b_f1, W_f2, b_f2).reshape(B, S)
    idx = _run_k2(imp)
    flat_idx = (idx + jnp.arange(B, dtype=i32)[:, None] * S).reshape(-1)
    x_sel = _run_k3(flat_idx, x2)
    positions, states, weights = _run_k4(
        x_sel, W_in, b_in, W_f1, b_f1, W_f2, b_f2, W_sp, b_sp)
    return (positions.reshape(B, NE, POS),
            states.reshape(B, NE, IN_DIM),
            weights.reshape(B, NE, 1))
